# Initial kernel scaffold; baseline (speedup 1.0000x reference)
import functools

import jax
import jax.numpy as jnp
from jax import lax
from jax.experimental import pallas as pl
from jax.experimental.pallas import tpu as pltpu
from jax.experimental.pallas import tpu_sc as plsc

N = 50000
E = 1600000
NUM_GRAPHS = 128

NC = 2
NS = 16
NW = NC * NS
L = 16

PER_W = E // NW
CHUNK = 2000
N_CHUNKS = PER_W // CHUNK
GROUPS = CHUNK // L


def _norm_newton(d2):
    xi = lax.bitcast_convert_type(d2, jnp.int32)
    yi = jnp.int32(0x5F3759DF) - (xi >> 1)
    y = lax.bitcast_convert_type(yi, jnp.float32)
    y = y * (1.5 - 0.5 * d2 * y * y)
    y = y * (1.5 - 0.5 * d2 * y * y)
    y = y * (1.5 - 0.5 * d2 * y * y)
    return d2 * y


@functools.partial(
    pl.kernel,
    mesh=plsc.VectorSubcoreMesh(core_axis_name="c", subcore_axis_name="s"),
    out_type=[
        jax.ShapeDtypeStruct((E, 3), jnp.float32),
        jax.ShapeDtypeStruct((E,), jnp.float32),
        jax.ShapeDtypeStruct((NW, NUM_GRAPHS), jnp.int32),
    ],
    scratch_types=[
        pltpu.VMEM((CHUNK,), jnp.int32),
        pltpu.VMEM((CHUNK,), jnp.int32),
        pltpu.VMEM((CHUNK, 4), jnp.float32),
        pltpu.VMEM((CHUNK, 4), jnp.float32),
        pltpu.VMEM((CHUNK, 3), jnp.float32),
        pltpu.VMEM((CHUNK,), jnp.float32),
        pltpu.VMEM((NUM_GRAPHS,), jnp.int32),
        pltpu.SemaphoreType.DMA,
    ],
)
def _edge_kernel(tab_hbm, src_hbm, dst_hbm, dv_out, dist_out, nb_out,
                 idx_j, idx_i, rows_j, rows_i, dv_buf, dist_buf, hist, sem):
    cid = lax.axis_index("c")
    sid = lax.axis_index("s")
    wid = sid * NC + cid

    zeros16 = jnp.zeros((L,), jnp.int32)
    for k in range(NUM_GRAPHS // L):
        hist[pl.ds(k * L, L)] = zeros16

    iota = lax.iota(jnp.int32, L)
    c0 = jnp.zeros((L,), jnp.int32)
    c1 = jnp.full((L,), 1, jnp.int32)
    c2 = jnp.full((L,), 2, jnp.int32)
    c3 = jnp.full((L,), 3, jnp.int32)
    ones_i = jnp.full((L,), 1, jnp.int32)

    def chunk_body(k, _):
        base = wid * PER_W + k * CHUNK
        pltpu.sync_copy(src_hbm.at[pl.ds(base, CHUNK)], idx_j)
        pltpu.sync_copy(dst_hbm.at[pl.ds(base, CHUNK)], idx_i)
        cp_j = pltpu.async_copy(tab_hbm.at[idx_j], rows_j, sem)
        cp_i = pltpu.async_copy(tab_hbm.at[idx_i], rows_i, sem)
        cp_j.wait()
        cp_i.wait()

        def group_body(g, _):
            b = g * L
            rid = b + iota
            jx = plsc.load_gather(rows_j, [rid, c0])
            jy = plsc.load_gather(rows_j, [rid, c1])
            jz = plsc.load_gather(rows_j, [rid, c2])
            ix = plsc.load_gather(rows_i, [rid, c0])
            iy = plsc.load_gather(rows_i, [rid, c1])
            iz = plsc.load_gather(rows_i, [rid, c2])
            gb = plsc.load_gather(rows_i, [rid, c3])
            dvx = jx - ix
            dvy = jy - iy
            dvz = jz - iz
            d2 = dvx * dvx + dvy * dvy + dvz * dvz
            dist_buf[pl.ds(b, L)] = _norm_newton(d2)
            plsc.store_scatter(dv_buf, [rid, c0], dvx)
            plsc.store_scatter(dv_buf, [rid, c1], dvy)
            plsc.store_scatter(dv_buf, [rid, c2], dvz)
            gidx = plsc.bitcast(gb, jnp.int32)
            plsc.addupdate_scatter(hist, [gidx], ones_i)
            return 0

        lax.fori_loop(0, GROUPS, group_body, 0, unroll=2)
        pltpu.sync_copy(dv_buf, dv_out.at[pl.ds(base, CHUNK)])
        pltpu.sync_copy(dist_buf, dist_out.at[pl.ds(base, CHUNK)])
        return 0

    lax.fori_loop(0, N_CHUNKS, chunk_body, 0)
    pltpu.sync_copy(hist, nb_out.at[wid])


def kernel(pos, edge_index, batch):
    src = edge_index[0]
    dst = edge_index[1]
    bbits = lax.bitcast_convert_type(batch, jnp.float32)[:, None]
    tab = jnp.concatenate([pos, bbits], axis=1)
    distance_vec, edge_dist, nb_parts = _edge_kernel(tab, src, dst)
    neighbors = jnp.sum(nb_parts, axis=0, dtype=jnp.int32)
    n_edges = edge_index.shape[1]
    cell_offsets = jnp.zeros((n_edges, 3), dtype=jnp.float32)
    cell_offset_distances = jnp.zeros((n_edges, 3), dtype=jnp.float32)
    return (edge_index, edge_dist, distance_vec, cell_offsets,
            cell_offset_distances, neighbors)


# speedup vs baseline: 11.5375x; 11.5375x over previous
import functools

import jax
import jax.numpy as jnp
from jax import lax
from jax.experimental import pallas as pl
from jax.experimental.pallas import tpu as pltpu
from jax.experimental.pallas import tpu_sc as plsc

N = 50000
E = 1600000
NUM_GRAPHS = 128

NC = 2
NS = 16
NW = NC * NS
L = 16

PER_W = E // NW
CHUNK = 2000
N_CHUNKS = PER_W // CHUNK
GROUPS = CHUNK // L


def _norm_newton(d2):
    xi = lax.bitcast_convert_type(d2, jnp.int32)
    yi = jnp.int32(0x5F3759DF) - (xi >> 1)
    y = lax.bitcast_convert_type(yi, jnp.float32)
    y = y * (1.5 - 0.5 * d2 * y * y)
    y = y * (1.5 - 0.5 * d2 * y * y)
    y = y * (1.5 - 0.5 * d2 * y * y)
    return d2 * y


@functools.partial(
    pl.kernel,
    mesh=plsc.VectorSubcoreMesh(core_axis_name="c", subcore_axis_name="s"),
    compiler_params=pltpu.CompilerParams(needs_layout_passes=False),
    out_type=[
        jax.ShapeDtypeStruct((3 * E,), jnp.float32),
        jax.ShapeDtypeStruct((E,), jnp.float32),
        jax.ShapeDtypeStruct((NW, NUM_GRAPHS), jnp.int32),
    ],
    scratch_types=[
        pltpu.VMEM((CHUNK,), jnp.int32),
        pltpu.VMEM((CHUNK,), jnp.int32),
        pltpu.VMEM((CHUNK,), jnp.float32),
        pltpu.VMEM((CHUNK,), jnp.float32),
        pltpu.VMEM((CHUNK,), jnp.float32),
        pltpu.VMEM((CHUNK,), jnp.float32),
        pltpu.VMEM((CHUNK,), jnp.float32),
        pltpu.VMEM((CHUNK,), jnp.float32),
        pltpu.VMEM((CHUNK,), jnp.int32),
        pltpu.VMEM((3 * CHUNK,), jnp.float32),
        pltpu.VMEM((CHUNK,), jnp.float32),
        pltpu.VMEM((NUM_GRAPHS,), jnp.int32),
        pltpu.SemaphoreType.DMA,
    ],
)
def _edge_kernel(tx_hbm, ty_hbm, tz_hbm, tb_hbm, src_hbm, dst_hbm,
                 dv_out, dist_out, nb_out,
                 idx_j, idx_i, jxv, jyv, jzv, ixv, iyv, izv, ibv,
                 dv_buf, dist_buf, hist, sem):
    cid = lax.axis_index("c")
    sid = lax.axis_index("s")
    wid = sid * NC + cid

    zeros16 = jnp.zeros((L,), jnp.int32)
    for k in range(NUM_GRAPHS // L):
        hist[pl.ds(k * L, L)] = zeros16

    iota = lax.iota(jnp.int32, L)
    iota3 = iota * 3
    ones_i = jnp.full((L,), 1, jnp.int32)

    def chunk_body(k, _):
        base = wid * PER_W + k * CHUNK
        pltpu.sync_copy(src_hbm.at[pl.ds(base, CHUNK)], idx_j)
        pltpu.sync_copy(dst_hbm.at[pl.ds(base, CHUNK)], idx_i)
        cps = [
            pltpu.async_copy(tx_hbm.at[idx_j], jxv, sem),
            pltpu.async_copy(ty_hbm.at[idx_j], jyv, sem),
            pltpu.async_copy(tz_hbm.at[idx_j], jzv, sem),
            pltpu.async_copy(tx_hbm.at[idx_i], ixv, sem),
            pltpu.async_copy(ty_hbm.at[idx_i], iyv, sem),
            pltpu.async_copy(tz_hbm.at[idx_i], izv, sem),
            pltpu.async_copy(tb_hbm.at[idx_i], ibv, sem),
        ]
        for cp in cps:
            cp.wait()

        def group_body(g, _):
            b = g * L
            dvx = jxv[pl.ds(b, L)] - ixv[pl.ds(b, L)]
            dvy = jyv[pl.ds(b, L)] - iyv[pl.ds(b, L)]
            dvz = jzv[pl.ds(b, L)] - izv[pl.ds(b, L)]
            d2 = dvx * dvx + dvy * dvy + dvz * dvz
            dist_buf[pl.ds(b, L)] = _norm_newton(d2)
            fid = b * 3 + iota3
            plsc.store_scatter(dv_buf, [fid], dvx)
            plsc.store_scatter(dv_buf, [fid + 1], dvy)
            plsc.store_scatter(dv_buf, [fid + 2], dvz)
            plsc.addupdate_scatter(hist, [ibv[pl.ds(b, L)]], ones_i)
            return 0

        lax.fori_loop(0, GROUPS, group_body, 0, unroll=2)
        pltpu.sync_copy(dv_buf, dv_out.at[pl.ds(3 * base, 3 * CHUNK)])
        pltpu.sync_copy(dist_buf, dist_out.at[pl.ds(base, CHUNK)])
        return 0

    lax.fori_loop(0, N_CHUNKS, chunk_body, 0)
    pltpu.sync_copy(hist, nb_out.at[wid])


def kernel(pos, edge_index, batch):
    src = edge_index[0]
    dst = edge_index[1]
    tx = pos[:, 0]
    ty = pos[:, 1]
    tz = pos[:, 2]
    dv_flat, edge_dist, nb_parts = _edge_kernel(tx, ty, tz, batch, src, dst)
    distance_vec = dv_flat.reshape(E, 3)
    neighbors = jnp.sum(nb_parts, axis=0, dtype=jnp.int32)
    n_edges = edge_index.shape[1]
    cell_offsets = jnp.zeros((n_edges, 3), dtype=jnp.float32)
    cell_offset_distances = jnp.zeros((n_edges, 3), dtype=jnp.float32)
    return (edge_index, edge_dist, distance_vec, cell_offsets,
            cell_offset_distances, neighbors)


# speedup vs baseline: 72.2293x; 6.2604x over previous
import functools

import jax
import jax.numpy as jnp
from jax import lax
from jax.experimental import pallas as pl
from jax.experimental.pallas import tpu as pltpu
from jax.experimental.pallas import tpu_sc as plsc

N = 50000
E = 1600000
NUM_GRAPHS = 128

NC = 2
NS = 16
NW = NC * NS
L = 16

MASK_HI = jnp.int32(-65536)
MASK_LO = jnp.int32(0xFFFF)


def _norm_newton(d2):
    xi = lax.bitcast_convert_type(d2, jnp.int32)
    yi = jnp.int32(0x5F3759DF) - (xi >> 1)
    y = lax.bitcast_convert_type(yi, jnp.float32)
    y = y * (1.5 - 0.5 * d2 * y * y)
    y = y * (1.5 - 0.5 * d2 * y * y)
    y = y * (1.5 - 0.5 * d2 * y * y)
    return d2 * y


def _unpack_hi(v):
    return lax.bitcast_convert_type(v & MASK_HI, jnp.float32)


def _unpack_lo_f(v):
    return lax.bitcast_convert_type(v << 16, jnp.float32)


def _make_edge_kernel(n_edges, chunk, n_chunks):
    per_w = n_edges // NW
    assert per_w == chunk * n_chunks and n_chunks % 2 == 1
    npair = (n_chunks - 1) // 2
    groups = chunk // L
    c2 = 2 * chunk

    @functools.partial(
        pl.kernel,
        mesh=plsc.VectorSubcoreMesh(core_axis_name="c", subcore_axis_name="s"),
        compiler_params=pltpu.CompilerParams(needs_layout_passes=False),
        out_type=[
            jax.ShapeDtypeStruct((n_edges,), jnp.float32),
            jax.ShapeDtypeStruct((n_edges,), jnp.float32),
            jax.ShapeDtypeStruct((n_edges,), jnp.float32),
            jax.ShapeDtypeStruct((n_edges,), jnp.float32),
            jax.ShapeDtypeStruct((NW, NUM_GRAPHS), jnp.int32),
        ],
        scratch_types=[
            pltpu.VMEM_SHARED((N,), jnp.int32),
            pltpu.VMEM_SHARED((N,), jnp.int32),
            pltpu.VMEM((c2,), jnp.int32), pltpu.VMEM((c2,), jnp.int32),
            pltpu.VMEM((c2,), jnp.int32), pltpu.VMEM((c2,), jnp.int32),
            pltpu.VMEM((c2,), jnp.int32), pltpu.VMEM((c2,), jnp.int32),
            pltpu.VMEM((chunk,), jnp.float32), pltpu.VMEM((chunk,), jnp.float32),
            pltpu.VMEM((chunk,), jnp.float32), pltpu.VMEM((chunk,), jnp.float32),
            pltpu.VMEM((chunk,), jnp.float32), pltpu.VMEM((chunk,), jnp.float32),
            pltpu.VMEM((chunk,), jnp.float32), pltpu.VMEM((chunk,), jnp.float32),
            pltpu.VMEM((NUM_GRAPHS,), jnp.int32),
            pltpu.SemaphoreType.DMA, pltpu.SemaphoreType.DMA,
            pltpu.SemaphoreType.DMA, pltpu.SemaphoreType.DMA,
            pltpu.SemaphoreType.DMA, pltpu.SemaphoreType.DMA,
        ],
    )
    def _edge_kernel(txy_hbm, tzb_hbm, src_hbm, dst_hbm,
                     dvx_out, dvy_out, dvz_out, dist_out, nb_out,
                     txy_sh, tzb_sh,
                     idxA, idxB, exyA, exyB, ezbA, ezbB,
                     oxA, oxB, oyA, oyB, ozA, ozB, odA, odB, hist,
                     semIA, semIB, semGA, semGB, semOA, semOB):
        cid = lax.axis_index("c")
        sid = lax.axis_index("s")
        wid = sid * NC + cid

        idx = [idxA, idxB]
        exy = [exyA, exyB]
        ezb = [ezbA, ezbB]
        ox = [oxA, oxB]
        oy = [oyA, oyB]
        oz = [ozA, ozB]
        od = [odA, odB]
        semI = [semIA, semIB]
        semG = [semGA, semGB]
        semO = [semOA, semOB]

        @pl.when(sid == 0)
        def _():
            pltpu.sync_copy(txy_hbm, txy_sh)
            pltpu.sync_copy(tzb_hbm, tzb_sh)

        plsc.subcore_barrier()

        zeros16 = jnp.zeros((L,), jnp.int32)
        for k in range(NUM_GRAPHS // L):
            hist[pl.ds(k * L, L)] = zeros16

        ones_i = jnp.full((L,), 1, jnp.int32)

        def start_idx(k, p):
            base = wid * per_w + k * chunk
            pltpu.async_copy(src_hbm.at[pl.ds(base, chunk)],
                             idx[p].at[pl.ds(0, chunk)], semI[p])
            pltpu.async_copy(dst_hbm.at[pl.ds(base, chunk)],
                             idx[p].at[pl.ds(chunk, chunk)], semI[p])

        def wait_idx(p):
            pltpu.make_async_copy(src_hbm.at[pl.ds(0, chunk)],
                                  idx[p].at[pl.ds(0, chunk)], semI[p]).wait()
            pltpu.make_async_copy(dst_hbm.at[pl.ds(0, chunk)],
                                  idx[p].at[pl.ds(chunk, chunk)], semI[p]).wait()

        def start_gathers(p):
            pltpu.async_copy(txy_sh.at[idx[p]], exy[p], semG[p])
            pltpu.async_copy(tzb_sh.at[idx[p]], ezb[p], semG[p])

        def wait_gathers(p):
            pltpu.make_async_copy(txy_sh.at[idx[p]], exy[p], semG[p]).wait()
            pltpu.make_async_copy(tzb_sh.at[idx[p]], ezb[p], semG[p]).wait()

        def start_out(k, p):
            base = wid * per_w + k * chunk
            pltpu.async_copy(ox[p], dvx_out.at[pl.ds(base, chunk)], semO[p])
            pltpu.async_copy(oy[p], dvy_out.at[pl.ds(base, chunk)], semO[p])
            pltpu.async_copy(oz[p], dvz_out.at[pl.ds(base, chunk)], semO[p])
            pltpu.async_copy(od[p], dist_out.at[pl.ds(base, chunk)], semO[p])

        def wait_out(p):
            pltpu.make_async_copy(ox[p], dvx_out.at[pl.ds(0, chunk)], semO[p]).wait()
            pltpu.make_async_copy(oy[p], dvy_out.at[pl.ds(0, chunk)], semO[p]).wait()
            pltpu.make_async_copy(oz[p], dvz_out.at[pl.ds(0, chunk)], semO[p]).wait()
            pltpu.make_async_copy(od[p], dist_out.at[pl.ds(0, chunk)], semO[p]).wait()

        def compute(p):
            def group_body(g, _):
                b = g * L
                vxyj = exy[p][pl.ds(b, L)]
                vzbj = ezb[p][pl.ds(b, L)]
                vxyi = exy[p][pl.ds(chunk + b, L)]
                vzbi = ezb[p][pl.ds(chunk + b, L)]
                dvx = _unpack_hi(vxyj) - _unpack_hi(vxyi)
                dvy = _unpack_lo_f(vxyj) - _unpack_lo_f(vxyi)
                dvz = _unpack_hi(vzbj) - _unpack_hi(vzbi)
                d2 = dvx * dvx + dvy * dvy + dvz * dvz
                ox[p][pl.ds(b, L)] = dvx
                oy[p][pl.ds(b, L)] = dvy
                oz[p][pl.ds(b, L)] = dvz
                od[p][pl.ds(b, L)] = _norm_newton(d2)
                plsc.addupdate_scatter(hist, [vzbi & MASK_LO], ones_i)
                return 0

            lax.fori_loop(0, groups, group_body, 0, unroll=4)

        start_idx(0, 0)
        wait_idx(0)
        start_gathers(0)
        start_idx(1, 1)

        def pair_body(m, _):
            k = 2 * m
            wait_gathers(0)
            wait_idx(1)
            start_gathers(1)
            start_idx(k + 2, 0)

            @pl.when(m > 0)
            def _():
                wait_out(0)
            compute(0)
            start_out(k, 0)

            wait_gathers(1)

            @pl.when(m < npair - 1)
            def _():
                start_idx(k + 3, 1)
            wait_idx(0)
            start_gathers(0)

            @pl.when(m > 0)
            def _():
                wait_out(1)
            compute(1)
            start_out(k + 1, 1)
            return 0

        lax.fori_loop(0, npair, pair_body, 0)

        wait_gathers(0)
        wait_out(0)
        compute(0)
        start_out(n_chunks - 1, 0)
        wait_out(0)
        wait_out(1)

        pltpu.sync_copy(hist, nb_out.at[wid])

    return _edge_kernel


EA = 960000
EB = E - EA
_kernel_a = _make_edge_kernel(EA, 2000, 15)
_kernel_b = _make_edge_kernel(EB, 4000, 5)


def kernel(pos, edge_index, batch):
    xu = lax.bitcast_convert_type(
        pos[:, 0].astype(jnp.bfloat16), jnp.uint16).astype(jnp.uint32)
    yu = lax.bitcast_convert_type(
        pos[:, 1].astype(jnp.bfloat16), jnp.uint16).astype(jnp.uint32)
    zu = lax.bitcast_convert_type(
        pos[:, 2].astype(jnp.bfloat16), jnp.uint16).astype(jnp.uint32)
    txy = lax.bitcast_convert_type((xu << 16) | yu, jnp.int32)
    tzb = lax.bitcast_convert_type(
        (zu << 16) | batch.astype(jnp.uint32), jnp.int32)
    ax, ay, az, ad, anb = _kernel_a(txy, tzb,
                                    edge_index[0, :EA], edge_index[1, :EA])
    bx, by, bz, bd, bnb = _kernel_b(txy, tzb,
                                    edge_index[0, EA:], edge_index[1, EA:])
    dva = jnp.stack([ax, ay, az], axis=1)
    dvb = jnp.stack([bx, by, bz], axis=1)
    distance_vec = jnp.concatenate([dva, dvb], axis=0)
    edge_dist = jnp.concatenate([ad, bd], axis=0)
    neighbors = jnp.sum(anb, axis=0, dtype=jnp.int32) + \
        jnp.sum(bnb, axis=0, dtype=jnp.int32)
    n_edges = edge_index.shape[1]
    cell_offsets = jnp.zeros((n_edges, 3), dtype=jnp.float32)
    cell_offset_distances = jnp.zeros((n_edges, 3), dtype=jnp.float32)
    return (edge_index, edge_dist, distance_vec, cell_offsets,
            cell_offset_distances, neighbors)


# speedup vs baseline: 78.1402x; 1.0818x over previous
import functools

import jax
import jax.numpy as jnp
from jax import lax
from jax.experimental import pallas as pl
from jax.experimental.pallas import tpu as pltpu
from jax.experimental.pallas import tpu_sc as plsc

N = 50000
E = 1600000
NUM_GRAPHS = 128

NC = 2
NS = 16
NW = NC * NS
L = 16

MASK_HI = jnp.int32(-65536)
MASK_LO = jnp.int32(0xFFFF)


def _norm_newton(d2):
    xi = lax.bitcast_convert_type(d2, jnp.int32)
    yi = jnp.int32(0x5F3759DF) - (xi >> 1)
    y = lax.bitcast_convert_type(yi, jnp.float32)
    y = y * (1.5 - 0.5 * d2 * y * y)
    y = y * (1.5 - 0.5 * d2 * y * y)
    return d2 * y


def _unpack_hi(v):
    return lax.bitcast_convert_type(v & MASK_HI, jnp.float32)


def _unpack_lo_f(v):
    return lax.bitcast_convert_type(v << 16, jnp.float32)


def _make_edge_kernel(n_edges, chunk, n_chunks):
    per_w = n_edges // NW
    assert per_w == chunk * n_chunks and n_chunks % 2 == 1
    npair = (n_chunks - 1) // 2
    groups = chunk // L
    c2 = 2 * chunk

    @functools.partial(
        pl.kernel,
        mesh=plsc.VectorSubcoreMesh(core_axis_name="c", subcore_axis_name="s"),
        compiler_params=pltpu.CompilerParams(needs_layout_passes=False),
        out_type=[
            jax.ShapeDtypeStruct((n_edges,), jnp.float32),
            jax.ShapeDtypeStruct((n_edges,), jnp.float32),
            jax.ShapeDtypeStruct((n_edges,), jnp.float32),
            jax.ShapeDtypeStruct((n_edges,), jnp.float32),
            jax.ShapeDtypeStruct((NW, NUM_GRAPHS), jnp.int32),
        ],
        scratch_types=[
            pltpu.VMEM_SHARED((N,), jnp.int32),
            pltpu.VMEM_SHARED((N,), jnp.int32),
            pltpu.VMEM((c2,), jnp.int32), pltpu.VMEM((c2,), jnp.int32),
            pltpu.VMEM((c2,), jnp.int32), pltpu.VMEM((c2,), jnp.int32),
            pltpu.VMEM((c2,), jnp.int32), pltpu.VMEM((c2,), jnp.int32),
            pltpu.VMEM((chunk,), jnp.float32), pltpu.VMEM((chunk,), jnp.float32),
            pltpu.VMEM((chunk,), jnp.float32), pltpu.VMEM((chunk,), jnp.float32),
            pltpu.VMEM((chunk,), jnp.float32), pltpu.VMEM((chunk,), jnp.float32),
            pltpu.VMEM((chunk,), jnp.float32), pltpu.VMEM((chunk,), jnp.float32),
            pltpu.VMEM((NUM_GRAPHS,), jnp.int32),
            pltpu.VMEM((NUM_GRAPHS,), jnp.int32),
            pltpu.VMEM((NUM_GRAPHS,), jnp.int32),
            pltpu.VMEM((NUM_GRAPHS,), jnp.int32),
            pltpu.SemaphoreType.DMA, pltpu.SemaphoreType.DMA,
            pltpu.SemaphoreType.DMA, pltpu.SemaphoreType.DMA,
            pltpu.SemaphoreType.DMA, pltpu.SemaphoreType.DMA,
        ],
    )
    def _edge_kernel(txy_hbm, tzb_hbm, src_hbm, dst_hbm,
                     dvx_out, dvy_out, dvz_out, dist_out, nb_out,
                     txy_sh, tzb_sh,
                     idxA, idxB, exyA, exyB, ezbA, ezbB,
                     oxA, oxB, oyA, oyB, ozA, ozB, odA, odB,
                     hist0, hist1, hist2, hist3,
                     semIA, semIB, semGA, semGB, semOA, semOB):
        cid = lax.axis_index("c")
        sid = lax.axis_index("s")
        wid = sid * NC + cid

        idx = [idxA, idxB]
        exy = [exyA, exyB]
        ezb = [ezbA, ezbB]
        ox = [oxA, oxB]
        oy = [oyA, oyB]
        oz = [ozA, ozB]
        od = [odA, odB]
        semI = [semIA, semIB]
        semG = [semGA, semGB]
        semO = [semOA, semOB]

        @pl.when(sid == 0)
        def _():
            pltpu.sync_copy(txy_hbm, txy_sh)
            pltpu.sync_copy(tzb_hbm, tzb_sh)

        plsc.subcore_barrier()

        hists = [hist0, hist1, hist2, hist3]
        zeros16 = jnp.zeros((L,), jnp.int32)
        for h in hists:
            for k in range(NUM_GRAPHS // L):
                h[pl.ds(k * L, L)] = zeros16

        ones_i = jnp.full((L,), 1, jnp.int32)

        def start_idx(k, p):
            base = wid * per_w + k * chunk
            pltpu.async_copy(src_hbm.at[pl.ds(base, chunk)],
                             idx[p].at[pl.ds(0, chunk)], semI[p])
            pltpu.async_copy(dst_hbm.at[pl.ds(base, chunk)],
                             idx[p].at[pl.ds(chunk, chunk)], semI[p])

        def wait_idx(p):
            pltpu.make_async_copy(src_hbm.at[pl.ds(0, chunk)],
                                  idx[p].at[pl.ds(0, chunk)], semI[p]).wait()
            pltpu.make_async_copy(dst_hbm.at[pl.ds(0, chunk)],
                                  idx[p].at[pl.ds(chunk, chunk)], semI[p]).wait()

        def start_gathers(p):
            pltpu.async_copy(txy_sh.at[idx[p]], exy[p], semG[p])
            pltpu.async_copy(tzb_sh.at[idx[p]], ezb[p], semG[p])

        def wait_gathers(p):
            pltpu.make_async_copy(txy_sh.at[idx[p]], exy[p], semG[p]).wait()
            pltpu.make_async_copy(tzb_sh.at[idx[p]], ezb[p], semG[p]).wait()

        def start_out(k, p):
            base = wid * per_w + k * chunk
            pltpu.async_copy(ox[p], dvx_out.at[pl.ds(base, chunk)], semO[p])
            pltpu.async_copy(oy[p], dvy_out.at[pl.ds(base, chunk)], semO[p])
            pltpu.async_copy(oz[p], dvz_out.at[pl.ds(base, chunk)], semO[p])
            pltpu.async_copy(od[p], dist_out.at[pl.ds(base, chunk)], semO[p])

        def wait_out(p):
            pltpu.make_async_copy(ox[p], dvx_out.at[pl.ds(0, chunk)], semO[p]).wait()
            pltpu.make_async_copy(oy[p], dvy_out.at[pl.ds(0, chunk)], semO[p]).wait()
            pltpu.make_async_copy(oz[p], dvz_out.at[pl.ds(0, chunk)], semO[p]).wait()
            pltpu.make_async_copy(od[p], dist_out.at[pl.ds(0, chunk)], semO[p]).wait()

        def compute(p):
            @plsc.parallel_loop(0, chunk, step=L, unroll=4)
            def _(b):
                vxyj = exy[p][pl.ds(b, L)]
                vzbj = ezb[p][pl.ds(b, L)]
                vxyi = exy[p][pl.ds(chunk + b, L)]
                vzbi = ezb[p][pl.ds(chunk + b, L)]
                dvx = _unpack_hi(vxyj) - _unpack_hi(vxyi)
                dvy = _unpack_lo_f(vxyj) - _unpack_lo_f(vxyi)
                dvz = _unpack_hi(vzbj) - _unpack_hi(vzbi)
                d2 = dvx * dvx + dvy * dvy + dvz * dvz
                ox[p][pl.ds(b, L)] = dvx
                oy[p][pl.ds(b, L)] = dvy
                oz[p][pl.ds(b, L)] = dvz
                od[p][pl.ds(b, L)] = _norm_newton(d2)

            def hist_body(g, _):
                b = 4 * g * L
                for q in range(4):
                    vzbi = ezb[p][pl.ds(chunk + b + q * L, L)]
                    plsc.addupdate_scatter(hists[q], [vzbi & MASK_LO], ones_i)
                return 0

            lax.fori_loop(0, groups // 4, hist_body, 0)
            for q in range(groups % 4):
                b = (groups // 4) * 4 * L + q * L
                vzbi = ezb[p][pl.ds(chunk + b, L)]
                plsc.addupdate_scatter(hists[q], [vzbi & MASK_LO], ones_i)

        start_idx(0, 0)
        wait_idx(0)
        start_gathers(0)
        start_idx(1, 1)

        def pair_body(m, _):
            k = 2 * m
            wait_gathers(0)
            wait_idx(1)
            start_gathers(1)
            start_idx(k + 2, 0)

            @pl.when(m > 0)
            def _():
                wait_out(0)
            compute(0)
            start_out(k, 0)

            wait_gathers(1)

            @pl.when(m < npair - 1)
            def _():
                start_idx(k + 3, 1)
            wait_idx(0)
            start_gathers(0)

            @pl.when(m > 0)
            def _():
                wait_out(1)
            compute(1)
            start_out(k + 1, 1)
            return 0

        lax.fori_loop(0, npair, pair_body, 0)

        wait_gathers(0)
        wait_out(0)
        compute(0)
        start_out(n_chunks - 1, 0)
        wait_out(0)
        wait_out(1)

        for k in range(NUM_GRAPHS // L):
            s = pl.ds(k * L, L)
            hist0[s] = hist0[s] + hist1[s] + hist2[s] + hist3[s]
        pltpu.sync_copy(hist0, nb_out.at[wid])

    return _edge_kernel


EA = 960000
EB = E - EA
_kernel_a = _make_edge_kernel(EA, 2000, 15)
_kernel_b = _make_edge_kernel(EB, 4000, 5)


def kernel(pos, edge_index, batch):
    xu = lax.bitcast_convert_type(
        pos[:, 0].astype(jnp.bfloat16), jnp.uint16).astype(jnp.uint32)
    yu = lax.bitcast_convert_type(
        pos[:, 1].astype(jnp.bfloat16), jnp.uint16).astype(jnp.uint32)
    zu = lax.bitcast_convert_type(
        pos[:, 2].astype(jnp.bfloat16), jnp.uint16).astype(jnp.uint32)
    txy = lax.bitcast_convert_type((xu << 16) | yu, jnp.int32)
    tzb = lax.bitcast_convert_type(
        (zu << 16) | batch.astype(jnp.uint32), jnp.int32)
    ax, ay, az, ad, anb = _kernel_a(txy, tzb,
                                    edge_index[0, :EA], edge_index[1, :EA])
    bx, by, bz, bd, bnb = _kernel_b(txy, tzb,
                                    edge_index[0, EA:], edge_index[1, EA:])
    dva = jnp.stack([ax, ay, az], axis=1)
    dvb = jnp.stack([bx, by, bz], axis=1)
    distance_vec = jnp.concatenate([dva, dvb], axis=0)
    edge_dist = jnp.concatenate([ad, bd], axis=0)
    neighbors = jnp.sum(anb, axis=0, dtype=jnp.int32) + \
        jnp.sum(bnb, axis=0, dtype=jnp.int32)
    n_edges = edge_index.shape[1]
    cell_offsets = jnp.zeros((n_edges, 3), dtype=jnp.float32)
    cell_offset_distances = jnp.zeros((n_edges, 3), dtype=jnp.float32)
    return (edge_index, edge_dist, distance_vec, cell_offsets,
            cell_offset_distances, neighbors)
